# baseline (speedup 1.0000x reference)
import functools

import jax
import jax.numpy as jnp
from jax.experimental import pallas as pl
from jax.experimental.pallas import tpu as pltpu

_NUM_Q = 4
_COMMIT_W = 0.25
_N_CHUNKS = 3


def _rvq_kernel(x_ref, cb_ref, q_ref, idx_ref, loss_ref,
                hi_ref, mid_ref, lo_ref, c2_ref, *, inv_count):
    i = pl.program_id(0)
    nsteps = pl.num_programs(0)

    @pl.when(i == 0)
    def _split():
        cb = cb_ref[...]
        hi = cb.astype(jnp.bfloat16)
        rem1 = cb - hi.astype(jnp.float32)
        mid = rem1.astype(jnp.bfloat16)
        rem2 = rem1 - mid.astype(jnp.float32)
        hi_ref[...] = hi
        mid_ref[...] = mid
        lo_ref[...] = rem2.astype(jnp.bfloat16)
        c2_ref[...] = jnp.sum(cb * cb, axis=2)
        loss_ref[0, 0] = jnp.float32(0.0)

    M = x_ref.shape[0]
    K = cb_ref.shape[1]
    NC = _N_CHUNKS
    H = M // NC
    lane_iota = jax.lax.broadcasted_iota(jnp.int32, (H, K), 1)
    rs = [x_ref[c * H:(c + 1) * H, :] for c in range(NC)]
    idx_cols = [[] for _ in range(NC)]
    loss_part = jnp.float32(0.0)
    for q in range(_NUM_Q):
        for h in range(NC):
            r = rs[h]
            r2 = jnp.sum(r * r, axis=1, keepdims=True)
            scores = jax.lax.dot_general(
                r.astype(jnp.bfloat16), hi_ref[q], (((1,), (1,)), ((), ())),
                preferred_element_type=jnp.float32)
            d2 = r2 - 2.0 * scores + c2_ref[q][None, :]
            minv = jnp.min(d2, axis=1, keepdims=True)
            idx = jnp.min(jnp.where(d2 == minv, lane_iota, K), axis=1)
            idx_cols[h].append(idx[:, None])
            onehot = jnp.where(lane_iota == idx[:, None],
                               jnp.float32(1),
                               jnp.float32(0)).astype(jnp.bfloat16)
            quant = jnp.float32(0.0)
            for part_ref in (hi_ref, mid_ref, lo_ref):
                quant = quant + jax.lax.dot_general(
                    onehot, part_ref[q], (((1,), (0,)), ((), ())),
                    preferred_element_type=jnp.float32)
            rs[h] = r - quant
            loss_part = loss_part + jnp.sum(rs[h] * rs[h])
    for c in range(NC):
        q_ref[c * H:(c + 1) * H, :] = x_ref[c * H:(c + 1) * H, :] - rs[c]
        idx_ref[c * H:(c + 1) * H, :] = jnp.concatenate(idx_cols[c], axis=1)

    loss_ref[0, 0] += loss_part

    @pl.when(i == nsteps - 1)
    def _finish():
        loss_ref[0, 0] = loss_ref[0, 0] * jnp.float32(_COMMIT_W * inv_count)


def _rvq_call(xs, cb, *, full_count):
    Ts, D = xs.shape
    K = cb.shape[1]
    M = 1152
    return pl.pallas_call(
        functools.partial(_rvq_kernel, inv_count=1.0 / full_count),
        grid=(Ts // M,),
        in_specs=[
            pl.BlockSpec((M, D), lambda i: (i, 0)),
            pl.BlockSpec((_NUM_Q, K, D), lambda i: (0, 0, 0)),
        ],
        out_specs=[
            pl.BlockSpec((M, D), lambda i: (i, 0)),
            pl.BlockSpec((M, _NUM_Q), lambda i: (i, 0)),
            pl.BlockSpec((1, 1), lambda i: (0, 0), memory_space=pltpu.SMEM),
        ],
        out_shape=[
            jax.ShapeDtypeStruct((Ts, D), jnp.float32),
            jax.ShapeDtypeStruct((Ts, _NUM_Q), jnp.int32),
            jax.ShapeDtypeStruct((1, 1), jnp.float32),
        ],
        scratch_shapes=[
            pltpu.VMEM((_NUM_Q, K, D), jnp.bfloat16),
            pltpu.VMEM((_NUM_Q, K, D), jnp.bfloat16),
            pltpu.VMEM((_NUM_Q, K, D), jnp.bfloat16),
            pltpu.VMEM((_NUM_Q, K), jnp.float32),
        ],
    )(xs, cb)


def kernel(x, codebooks):
    B, S, D = x.shape
    T = B * S
    qf, idxf, loss = _rvq_call(x.reshape(T, D), codebooks,
                               full_count=float(T * D))
    return qf.reshape(B, S, D), idxf.reshape(B, S, _NUM_Q), loss[0, 0]


# speedup vs baseline: 1.0486x; 1.0486x over previous
import functools

import jax
import jax.numpy as jnp
from jax.experimental import pallas as pl
from jax.experimental.pallas import tpu as pltpu

_NUM_Q = 4
_COMMIT_W = 0.25
_N_CHUNKS = 2


def _rvq_kernel(x_ref, cb_ref, q_ref, idx_ref, loss_ref,
                hi_ref, mid_ref, lo_ref, c2_ref, *, inv_count):
    i = pl.program_id(0)
    nsteps = pl.num_programs(0)

    @pl.when(i == 0)
    def _split():
        cb = cb_ref[...]
        hi = cb.astype(jnp.bfloat16)
        rem1 = cb - hi.astype(jnp.float32)
        mid = rem1.astype(jnp.bfloat16)
        rem2 = rem1 - mid.astype(jnp.float32)
        hi_ref[...] = hi
        mid_ref[...] = mid
        lo_ref[...] = rem2.astype(jnp.bfloat16)
        c2_ref[...] = jnp.sum(cb * cb, axis=2)
        loss_ref[0, 0] = jnp.float32(0.0)

    M = x_ref.shape[0]
    K = cb_ref.shape[1]
    NC = _N_CHUNKS
    H = M // NC
    lane_iota = jax.lax.broadcasted_iota(jnp.int32, (H, K), 1)
    rs = [x_ref[c * H:(c + 1) * H, :] for c in range(NC)]
    idx_cols = [[] for _ in range(NC)]
    loss_part = jnp.float32(0.0)
    for q in range(_NUM_Q):
        for h in range(NC):
            r = rs[h]
            r2 = jnp.sum(r * r, axis=1, keepdims=True)
            scores = jax.lax.dot_general(
                r.astype(jnp.bfloat16), hi_ref[q], (((1,), (1,)), ((), ())),
                preferred_element_type=jnp.float32)
            d2 = r2 - 2.0 * scores + c2_ref[q][None, :]
            minv = jnp.min(d2, axis=1, keepdims=True)
            idx = jnp.min(jnp.where(d2 == minv, lane_iota, K), axis=1)
            idx_cols[h].append(idx[:, None])
            onehot = jnp.where(lane_iota == idx[:, None],
                               jnp.float32(1),
                               jnp.float32(0)).astype(jnp.bfloat16)
            quant = jnp.float32(0.0)
            for part_ref in (hi_ref, mid_ref, lo_ref):
                quant = quant + jax.lax.dot_general(
                    onehot, part_ref[q], (((1,), (0,)), ((), ())),
                    preferred_element_type=jnp.float32)
            rs[h] = r - quant
            loss_part = loss_part + jnp.sum(rs[h] * rs[h])
    for c in range(NC):
        q_ref[c * H:(c + 1) * H, :] = x_ref[c * H:(c + 1) * H, :] - rs[c]
        idx_ref[c * H:(c + 1) * H, :] = jnp.concatenate(idx_cols[c], axis=1)

    loss_ref[0, 0] += loss_part

    @pl.when(i == nsteps - 1)
    def _finish():
        loss_ref[0, 0] = loss_ref[0, 0] * jnp.float32(_COMMIT_W * inv_count)


def _rvq_call(xs, cb, *, full_count):
    Ts, D = xs.shape
    K = cb.shape[1]
    M = 2304
    return pl.pallas_call(
        functools.partial(_rvq_kernel, inv_count=1.0 / full_count),
        grid=(Ts // M,),
        in_specs=[
            pl.BlockSpec((M, D), lambda i: (i, 0)),
            pl.BlockSpec((_NUM_Q, K, D), lambda i: (0, 0, 0)),
        ],
        out_specs=[
            pl.BlockSpec((M, D), lambda i: (i, 0)),
            pl.BlockSpec((M, _NUM_Q), lambda i: (i, 0)),
            pl.BlockSpec((1, 1), lambda i: (0, 0), memory_space=pltpu.SMEM),
        ],
        out_shape=[
            jax.ShapeDtypeStruct((Ts, D), jnp.float32),
            jax.ShapeDtypeStruct((Ts, _NUM_Q), jnp.int32),
            jax.ShapeDtypeStruct((1, 1), jnp.float32),
        ],
        scratch_shapes=[
            pltpu.VMEM((_NUM_Q, K, D), jnp.bfloat16),
            pltpu.VMEM((_NUM_Q, K, D), jnp.bfloat16),
            pltpu.VMEM((_NUM_Q, K, D), jnp.bfloat16),
            pltpu.VMEM((_NUM_Q, K), jnp.float32),
        ],
    )(xs, cb)


def kernel(x, codebooks):
    B, S, D = x.shape
    T = B * S
    qf, idxf, loss = _rvq_call(x.reshape(T, D), codebooks,
                               full_count=float(T * D))
    return qf.reshape(B, S, D), idxf.reshape(B, S, _NUM_Q), loss[0, 0]
